# baseline (speedup 1.0000x reference)
import functools

import jax
import jax.numpy as jnp
from jax import lax
from jax.experimental import pallas as pl
from jax.experimental.pallas import tpu as pltpu
from jax.experimental.pallas import tpu_sc as plsc

N_NODES = 10000
N_EDGES = 320000
IN_DIM = 128
HID = 16
HEADS = 8
F = HEADS * HID
N_TGT = 5000

NC = 2
NS = 16

EPT = N_EDGES // NS
MC = 80
NMC = EPT // MC
IW = 3 * MC

T_PAD = 5120
TPT = T_PAD // NS
NTR = TPT // MC

NZC = N_NODES // MC

_BLK = 1000
_NB = N_NODES // _BLK


def _tc_body(x_ref, w_ref, al_ref, ar_ref, hel_ref, er_ref, m_ref):
    i = pl.program_id(1)
    h = jnp.dot(x_ref[0], w_ref[0], preferred_element_type=jnp.float32)
    el = jnp.dot(h, al_ref[0], preferred_element_type=jnp.float32)
    er = jnp.dot(h, ar_ref[0], preferred_element_type=jnp.float32)
    hel_ref[0] = jnp.concatenate([h, el, el], axis=1)
    er_ref[0] = jnp.concatenate([er, er], axis=1)
    cur = jnp.concatenate(
        [jnp.max(el, axis=0, keepdims=True), jnp.max(er, axis=0, keepdims=True)],
        axis=1)

    @pl.when(i == 0)
    def _():
        m_ref[0] = cur

    @pl.when(i > 0)
    def _():
        m_ref[0] = jnp.maximum(m_ref[0], cur)

    @pl.when(i == _NB - 1)
    def _():
        acc = m_ref[0]
        s = acc[:, 0:HEADS] + acc[:, HEADS:2 * HEADS]
        mf = jnp.where(s > 0.0, s, 0.2 * s)
        m_ref[0] = jnp.concatenate([mf, mf], axis=1)


def _tc_call(x_all, w_all, al, ar):
    return pl.pallas_call(
        _tc_body,
        grid=(2, _NB),
        in_specs=[
            pl.BlockSpec((1, _BLK, IN_DIM), lambda m, i: (m, i, 0)),
            pl.BlockSpec((1, IN_DIM, F), lambda m, i: (m, 0, 0)),
            pl.BlockSpec((1, F, HEADS), lambda m, i: (m, 0, 0)),
            pl.BlockSpec((1, F, HEADS), lambda m, i: (m, 0, 0)),
        ],
        out_specs=[
            pl.BlockSpec((1, _BLK, F + 16), lambda m, i: (m, i, 0)),
            pl.BlockSpec((1, _BLK, 16), lambda m, i: (m, i, 0)),
            pl.BlockSpec((1, 1, 16), lambda m, i: (m, 0, 0)),
        ],
        out_shape=[
            jax.ShapeDtypeStruct((2, N_NODES, F + 16), jnp.float32),
            jax.ShapeDtypeStruct((2, N_NODES, 16), jnp.float32),
            jax.ShapeDtypeStruct((2, 1, 16), jnp.float32),
        ],
    )(x_all, w_all, al, ar)


def _sc_body(hel2, era, eidx, tgts, mvec, bias2, out,
             acc_sp, den_sp,
             ix0, ix1, hl0, hl1, er0, er1, hgs, exb, mv, bias_v,
             gs0, gs1):
    cid = lax.axis_index("c")
    sid = lax.axis_index("s")

    pltpu.sync_copy(mvec.at[pl.ds(16 * cid, 16)], mv)
    pltpu.sync_copy(bias2.at[pl.ds(F * cid, F)], bias_v)

    @pl.loop(0, MC)
    def _(c):
        exb[c, :] = jnp.zeros((16,), jnp.float32)
        for j in range(F // 16):
            hgs[c, pl.ds(j * 16, 16)] = jnp.zeros((16,), jnp.float32)

    for r in range(8):
        ck = sid * 8 + r

        @pl.when(ck < NZC)
        def _():
            rows = pl.ds(ck * MC, MC)
            pltpu.sync_copy(hgs, acc_sp.at[rows])
            pltpu.sync_copy(exb, den_sp.at[rows])

    plsc.subcore_barrier()

    ibase0 = (cid * NS + sid) * (NMC * IW)

    def load_idx(c, ix, gs):
        base = ibase0 + c * IW
        for r in range(3):
            pltpu.async_copy(eidx.at[pl.ds(base + r * MC, MC)], ix.at[r], gs)
        for r in range(3):
            pltpu.make_async_copy(eidx.at[pl.ds(0, MC)], ix.at[r], gs).wait()

    def fire_gathers(ix, hl, er, gs):
        pltpu.async_copy(era.at[ix.at[1]], er, gs)
        pltpu.async_copy(hel2.at[ix.at[0]], hl, gs)

    def wait_gathers(hl, er, gs):
        pltpu.make_async_copy(era.at[pl.ds(0, MC)], er, gs).wait()
        pltpu.make_async_copy(hel2.at[pl.ds(0, MC)], hl, gs).wait()

    def compute_scatter(ix, hl, er):
        @pl.loop(0, MC)
        def _(c):
            e = hl[c, pl.ds(F, 16)] + er[c, :]
            e = jnp.where(e > 0.0, e, 0.2 * e)
            ex = jnp.exp(e - mv[...])
            exb[c, :] = ex
            for hh in range(HEADS):
                sl = pl.ds(hh * HID, HID)
                hgs[c, sl] = hl[c, sl] * ex[hh]

        pltpu.sync_copy(exb, den_sp.at[ix.at[2]], add=True)
        pltpu.sync_copy(hgs, acc_sp.at[ix.at[2]], add=True)

    bufs = ((ix0, hl0, er0, gs0), (ix1, hl1, er1, gs1))

    for j in range(2):
        ix, hl, er, gs = bufs[j]
        load_idx(j, ix, gs)
        fire_gathers(ix, hl, er, gs)

    @pl.loop(0, NMC - 2, step=2)
    def _(g):
        for j in range(2):
            ix, hl, er, gs = bufs[j]
            wait_gathers(hl, er, gs)
            compute_scatter(ix, hl, er)
            load_idx(g + j + 2, ix, gs)
            fire_gathers(ix, hl, er, gs)

    for j in range(2):
        ix, hl, er, gs = bufs[j]
        wait_gathers(hl, er, gs)
        compute_scatter(ix, hl, er)

    plsc.subcore_barrier()

    tb = sid * TPT
    tgt_base = cid * T_PAD + tb
    for r in range(NTR):
        rb = r * MC
        pltpu.sync_copy(tgts.at[pl.ds(tgt_base + rb, MC)], ix0.at[0])
        pltpu.sync_copy(acc_sp.at[ix0.at[0]], hgs)
        pltpu.sync_copy(den_sp.at[ix0.at[0]], exb)

        @pl.loop(0, MC)
        def _(t):
            dv = jnp.maximum(exb[t, :], 1e-9)
            for hh in range(HEADS):
                sl = pl.ds(hh * HID, HID)
                v = hgs[t, sl] / dv[hh] + bias_v[sl]
                v = jnp.where(v > 0.0, v, jnp.exp(v) - 1.0)
                hgs[t, sl] = v

        pltpu.sync_copy(hgs, out.at[cid, pl.ds(tb + rb, MC)])


def _sc_call(hel2, era, eidx, tgts, mvec, bias2):
    mesh = plsc.VectorSubcoreMesh(core_axis_name="c", subcore_axis_name="s")
    kfn = pl.kernel(
        _sc_body,
        out_type=jax.ShapeDtypeStruct((2, T_PAD, F), jnp.float32),
        mesh=mesh,
        compiler_params=pltpu.CompilerParams(use_tc_tiling_on_sc=False),
        scratch_types=[
            pltpu.VMEM_SHARED((N_NODES, F), jnp.float32),
            pltpu.VMEM_SHARED((N_NODES, 16), jnp.float32),
            pltpu.VMEM((3, MC), jnp.int32),
            pltpu.VMEM((3, MC), jnp.int32),
            pltpu.VMEM((MC, F + 16), jnp.float32),
            pltpu.VMEM((MC, F + 16), jnp.float32),
            pltpu.VMEM((MC, 16), jnp.float32),
            pltpu.VMEM((MC, 16), jnp.float32),
            pltpu.VMEM((MC, F), jnp.float32),
            pltpu.VMEM((MC, 16), jnp.float32),
            pltpu.VMEM((16,), jnp.float32),
            pltpu.VMEM((F,), jnp.float32),
            pltpu.SemaphoreType.DMA,
            pltpu.SemaphoreType.DMA,
        ],
    )
    return kfn(hel2, era, eidx, tgts, mvec, bias2)


def _attn_mat(a):
    eye = jnp.eye(HEADS, dtype=jnp.float32)
    return (a[:, :, None] * eye[:, None, :]).reshape(F, HEADS)


def _idx_stream(edge_index, m):
    s = edge_index[0].astype(jnp.int32) + m * N_NODES
    d = edge_index[1].astype(jnp.int32)
    arr = jnp.stack(
        [s.reshape(NS, NMC, MC),
         (d + m * N_NODES).reshape(NS, NMC, MC),
         d.reshape(NS, NMC, MC)],
        axis=2)
    return arr.reshape(-1)


def kernel(x_0, x_1, edge_index_0, edge_index_1, target_idx_0, target_idx_1,
           W_0, attn_l_0, attn_r_0, b_0, W_1, attn_l_1, attn_r_1, b_1):
    x_all = jnp.stack([x_0, x_1])
    w_all = jnp.stack([W_0, W_1])
    al = jnp.stack([_attn_mat(attn_l_0), _attn_mat(attn_l_1)])
    ar = jnp.stack([_attn_mat(attn_r_0), _attn_mat(attn_r_1)])

    hel3, er3, m3 = _tc_call(x_all, w_all, al, ar)
    hel2 = hel3.reshape(2 * N_NODES, F + 16)
    era = er3.reshape(2 * N_NODES, 16)
    mvec = m3.reshape(32)

    eidx = jnp.concatenate([_idx_stream(edge_index_0, 0),
                            _idx_stream(edge_index_1, 1)])
    pad = jnp.zeros((T_PAD - N_TGT,), jnp.int32)
    tgts = jnp.concatenate([
        target_idx_0.astype(jnp.int32), pad,
        target_idx_1.astype(jnp.int32), pad,
    ])
    bias2 = jnp.concatenate([b_0, b_1])

    out = _sc_call(hel2, era, eidx, tgts, mvec, bias2)
    return (out[0, :N_TGT], out[1, :N_TGT])


# speedup vs baseline: 1.6660x; 1.6660x over previous
import functools

import jax
import jax.numpy as jnp
from jax import lax
from jax.experimental import pallas as pl
from jax.experimental.pallas import tpu as pltpu
from jax.experimental.pallas import tpu_sc as plsc

N_NODES = 10000
N_EDGES = 320000
IN_DIM = 128
HID = 16
HEADS = 8
F = HEADS * HID
N_TGT = 5000

NC = 2
NS = 16

EPT = N_EDGES // NS
MC = 80
NMC = EPT // MC
IW = 3 * MC

T_PAD = 5120
TPT = T_PAD // NS
NTR = TPT // MC

NZC = N_NODES // MC

_BLK = 1000
_NB = N_NODES // _BLK


def _tc_body(x_ref, w_ref, al_ref, ar_ref, h_ref, el_ref, er_ref, m_ref):
    i = pl.program_id(1)
    h = jnp.dot(x_ref[0], w_ref[0], preferred_element_type=jnp.float32)
    el = jnp.dot(h, al_ref[0], preferred_element_type=jnp.float32)
    er = jnp.dot(h, ar_ref[0], preferred_element_type=jnp.float32)
    h_ref[0] = h
    el_ref[0] = jnp.concatenate([el, el], axis=1)
    er_ref[0] = jnp.concatenate([er, er], axis=1)
    cur = jnp.concatenate(
        [jnp.max(el, axis=0, keepdims=True), jnp.max(er, axis=0, keepdims=True)],
        axis=1)

    @pl.when(i == 0)
    def _():
        m_ref[0] = cur

    @pl.when(i > 0)
    def _():
        m_ref[0] = jnp.maximum(m_ref[0], cur)

    @pl.when(i == _NB - 1)
    def _():
        acc = m_ref[0]
        s = acc[:, 0:HEADS] + acc[:, HEADS:2 * HEADS]
        mf = jnp.where(s > 0.0, s, 0.2 * s)
        m_ref[0] = jnp.concatenate([mf, mf], axis=1)


def _tc_call(x_all, w_all, al, ar):
    return pl.pallas_call(
        _tc_body,
        grid=(2, _NB),
        in_specs=[
            pl.BlockSpec((1, _BLK, IN_DIM), lambda m, i: (m, i, 0)),
            pl.BlockSpec((1, IN_DIM, F), lambda m, i: (m, 0, 0)),
            pl.BlockSpec((1, F, HEADS), lambda m, i: (m, 0, 0)),
            pl.BlockSpec((1, F, HEADS), lambda m, i: (m, 0, 0)),
        ],
        out_specs=[
            pl.BlockSpec((1, _BLK, F), lambda m, i: (m, i, 0)),
            pl.BlockSpec((1, _BLK, 16), lambda m, i: (m, i, 0)),
            pl.BlockSpec((1, _BLK, 16), lambda m, i: (m, i, 0)),
            pl.BlockSpec((1, 1, 16), lambda m, i: (m, 0, 0)),
        ],
        out_shape=[
            jax.ShapeDtypeStruct((2, N_NODES, F), jnp.float32),
            jax.ShapeDtypeStruct((2, N_NODES, 16), jnp.float32),
            jax.ShapeDtypeStruct((2, N_NODES, 16), jnp.float32),
            jax.ShapeDtypeStruct((2, 1, 16), jnp.float32),
        ],
    )(x_all, w_all, al, ar)


def _sc_body(h2, ela, era, eidx, tgts, mvec, bias2, out,
             acc_sp, den_sp,
             ix0, ix1, hg0, hg1, el0, el1, er0, er1, exb, mv, bias_v,
             gs0, gs1, es0, es1):
    cid = lax.axis_index("c")
    sid = lax.axis_index("s")

    pltpu.sync_copy(mvec.at[pl.ds(16 * cid, 16)], mv)
    pltpu.sync_copy(bias2.at[pl.ds(F * cid, F)], bias_v)

    @pl.loop(0, MC)
    def _(c):
        exb[c, :] = jnp.zeros((16,), jnp.float32)
        for j in range(F // 16):
            hg0[c, pl.ds(j * 16, 16)] = jnp.zeros((16,), jnp.float32)

    for r in range(8):
        ck = sid * 8 + r

        @pl.when(ck < NZC)
        def _():
            rows = pl.ds(ck * MC, MC)
            pltpu.sync_copy(hg0, acc_sp.at[rows])
            pltpu.sync_copy(exb, den_sp.at[rows])

    plsc.subcore_barrier()

    ibase0 = (cid * NS + sid) * (NMC * IW)

    def load_idx(c, ix, gs):
        base = ibase0 + c * IW
        for r in range(3):
            pltpu.async_copy(eidx.at[pl.ds(base + r * MC, MC)], ix.at[r], gs)
        for r in range(3):
            pltpu.make_async_copy(eidx.at[pl.ds(0, MC)], ix.at[r], gs).wait()

    def fire_gathers(ix, hg, el, er, gs, es):
        pltpu.async_copy(ela.at[ix.at[0]], el, es)
        pltpu.async_copy(era.at[ix.at[1]], er, es)
        pltpu.async_copy(h2.at[ix.at[0]], hg, gs)

    def compute_scatter(ix, hg, el, er, gs, es):
        pltpu.make_async_copy(ela.at[pl.ds(0, MC)], el, es).wait()
        pltpu.make_async_copy(era.at[pl.ds(0, MC)], er, es).wait()

        @pl.loop(0, MC)
        def _(c):
            e = el[c, :] + er[c, :]
            e = jnp.where(e > 0.0, e, 0.2 * e)
            exb[c, :] = jnp.exp(e - mv[...])

        pltpu.make_async_copy(h2.at[pl.ds(0, MC)], hg, gs).wait()

        @pl.loop(0, MC)
        def _(c):
            ex = exb[c, :]
            for hh in range(HEADS):
                sl = pl.ds(hh * HID, HID)
                hg[c, sl] = hg[c, sl] * ex[hh]

        pltpu.sync_copy(exb, den_sp.at[ix.at[2]], add=True)
        pltpu.sync_copy(hg, acc_sp.at[ix.at[2]], add=True)

    bufs = ((ix0, hg0, el0, er0, gs0, es0), (ix1, hg1, el1, er1, gs1, es1))

    for j in range(2):
        ix, hg, el, er, gs, es = bufs[j]
        load_idx(j, ix, gs)
        fire_gathers(ix, hg, el, er, gs, es)

    @pl.loop(0, NMC - 2, step=2)
    def _(g):
        for j in range(2):
            ix, hg, el, er, gs, es = bufs[j]
            compute_scatter(ix, hg, el, er, gs, es)
            load_idx(g + j + 2, ix, gs)
            fire_gathers(ix, hg, el, er, gs, es)

    for j in range(2):
        ix, hg, el, er, gs, es = bufs[j]
        compute_scatter(ix, hg, el, er, gs, es)

    plsc.subcore_barrier()

    tb = sid * TPT
    tgt_base = cid * T_PAD + tb
    for r in range(NTR):
        rb = r * MC
        pltpu.sync_copy(tgts.at[pl.ds(tgt_base + rb, MC)], ix0.at[0])
        pltpu.sync_copy(acc_sp.at[ix0.at[0]], hg0)
        pltpu.sync_copy(den_sp.at[ix0.at[0]], exb)

        @pl.loop(0, MC)
        def _(t):
            dv = jnp.maximum(exb[t, :], 1e-9)
            for hh in range(HEADS):
                sl = pl.ds(hh * HID, HID)
                v = hg0[t, sl] / dv[hh] + bias_v[sl]
                v = jnp.where(v > 0.0, v, jnp.exp(v) - 1.0)
                hg0[t, sl] = v

        pltpu.sync_copy(hg0, out.at[cid, pl.ds(tb + rb, MC)])


def _sc_call(h2, ela, era, eidx, tgts, mvec, bias2):
    mesh = plsc.VectorSubcoreMesh(core_axis_name="c", subcore_axis_name="s")
    kfn = pl.kernel(
        _sc_body,
        out_type=jax.ShapeDtypeStruct((2, T_PAD, F), jnp.float32),
        mesh=mesh,
        compiler_params=pltpu.CompilerParams(use_tc_tiling_on_sc=False),
        scratch_types=[
            pltpu.VMEM_SHARED((N_NODES, F), jnp.float32),
            pltpu.VMEM_SHARED((N_NODES, 16), jnp.float32),
            pltpu.VMEM((3, MC), jnp.int32),
            pltpu.VMEM((3, MC), jnp.int32),
            pltpu.VMEM((MC, F), jnp.float32),
            pltpu.VMEM((MC, F), jnp.float32),
            pltpu.VMEM((MC, 16), jnp.float32),
            pltpu.VMEM((MC, 16), jnp.float32),
            pltpu.VMEM((MC, 16), jnp.float32),
            pltpu.VMEM((MC, 16), jnp.float32),
            pltpu.VMEM((MC, 16), jnp.float32),
            pltpu.VMEM((16,), jnp.float32),
            pltpu.VMEM((F,), jnp.float32),
            pltpu.SemaphoreType.DMA,
            pltpu.SemaphoreType.DMA,
            pltpu.SemaphoreType.DMA,
            pltpu.SemaphoreType.DMA,
        ],
    )
    return kfn(h2, ela, era, eidx, tgts, mvec, bias2)


def _attn_mat(a):
    eye = jnp.eye(HEADS, dtype=jnp.float32)
    return (a[:, :, None] * eye[:, None, :]).reshape(F, HEADS)


def _idx_stream(edge_index, m):
    s = edge_index[0].astype(jnp.int32) + m * N_NODES
    d = edge_index[1].astype(jnp.int32)
    arr = jnp.stack(
        [s.reshape(NS, NMC, MC),
         (d + m * N_NODES).reshape(NS, NMC, MC),
         d.reshape(NS, NMC, MC)],
        axis=2)
    return arr.reshape(-1)


def kernel(x_0, x_1, edge_index_0, edge_index_1, target_idx_0, target_idx_1,
           W_0, attn_l_0, attn_r_0, b_0, W_1, attn_l_1, attn_r_1, b_1):
    x_all = jnp.stack([x_0, x_1])
    w_all = jnp.stack([W_0, W_1])
    al = jnp.stack([_attn_mat(attn_l_0), _attn_mat(attn_l_1)])
    ar = jnp.stack([_attn_mat(attn_r_0), _attn_mat(attn_r_1)])

    h3, el3, er3, m3 = _tc_call(x_all, w_all, al, ar)
    h2 = h3.reshape(2 * N_NODES, F)
    ela = el3.reshape(2 * N_NODES, 16)
    era = er3.reshape(2 * N_NODES, 16)
    mvec = m3.reshape(32)

    eidx = jnp.concatenate([_idx_stream(edge_index_0, 0),
                            _idx_stream(edge_index_1, 1)])
    pad = jnp.zeros((T_PAD - N_TGT,), jnp.int32)
    tgts = jnp.concatenate([
        target_idx_0.astype(jnp.int32), pad,
        target_idx_1.astype(jnp.int32), pad,
    ])
    bias2 = jnp.concatenate([b_0, b_1])

    out = _sc_call(h2, ela, era, eidx, tgts, mvec, bias2)
    return (out[0, :N_TGT], out[1, :N_TGT])
